# baseline (speedup 1.0000x reference)
import functools

import numpy as np

import jax
import jax.numpy as jnp
from jax.experimental import pallas as pl
from jax.experimental.pallas import tpu as pltpu

F = 51
FP = 64
K = 128
BINS = 2
G = 2
T = 16384
R = T // G
CG = 2 * G * FP
KG = G * K


def _band_x(Wp0, Wp1, ksize, ctr, bias):
    zero = jnp.zeros((FP, K), Wp0.dtype)
    oblocks = []
    for o in (-1, 0, 1):
        cols = []
        for s in range(G):
            rows = []
            for part, Wp in ((0, Wp0), (1, Wp1)):
                for r in range(G):
                    t = G * o + r - s + ctr
                    if 0 <= t < ksize:
                        piece = Wp[t]
                        if part == 0 and o == 0 and r == s:
                            piece = piece.at[F].set(bias)
                    else:
                        piece = zero
                    rows.append(piece)
            cols.append(jnp.concatenate(rows, axis=0))
        oblocks.append(jnp.concatenate(cols, axis=1))
    return jnp.concatenate(oblocks, axis=0)


def _band_y(Wy, ksize, ctr):
    zero = jnp.zeros((K, K), Wy.dtype)
    oblocks = []
    for o in (-1, 0, 1):
        cols = []
        for s in range(G):
            rows = []
            for r in range(G):
                t = G * o + r - s + ctr
                rows.append(Wy[t] if 0 <= t < ksize else zero)
            cols.append(jnp.concatenate(rows, axis=0))
        oblocks.append(jnp.concatenate(cols, axis=1))
    return jnp.concatenate(oblocks, axis=0)


def _head_w(Whp0, Whp1, Why, bias):
    zf = jnp.zeros((FP, BINS), Whp0.dtype)
    zy = jnp.zeros((K, BINS), Why.dtype)
    cols = []
    for s in range(G):
        rows = []
        for part, Wp in ((0, Whp0), (1, Whp1)):
            for r in range(G):
                if r == s:
                    piece = Wp.at[F].set(bias) if part == 0 else Wp
                else:
                    piece = zf
                rows.append(piece)
        for r in range(G):
            rows.append(Why if r == s else zy)
        cols.append(jnp.concatenate(rows, axis=0))
    return jnp.concatenate(cols, axis=1)


def _fused_kernel(prev_ref, cur_ref, next_ref,
                  w0b_ref, w1b_ref, whb_ref,
                  out_ref):
    i = pl.program_id(0)
    nb = pl.num_programs(0)
    f32 = jnp.float32
    cdt = cur_ref.dtype
    xe = jnp.concatenate(
        [prev_ref[R - 2:, :], cur_ref[...], next_ref[:2, :]], axis=0)
    ng = nb * R
    ge = i * R - 2 + jax.lax.broadcasted_iota(jnp.int32, (R + 4, 1), 0)
    xe = jnp.where((ge >= 0) & (ge < ng), xe, 0.0)
    xc = jnp.concatenate([xe, jnp.maximum(xe, 0.0)], axis=1)

    acc0 = (jnp.dot(xc[0:R + 2], w0b_ref[0:CG], preferred_element_type=f32)
            + jnp.dot(xc[1:R + 3], w0b_ref[CG:2 * CG],
                      preferred_element_type=f32)
            + jnp.dot(xc[2:R + 4], w0b_ref[2 * CG:3 * CG],
                      preferred_element_type=f32))
    y0 = jnp.maximum(acc0, 0.0).astype(cdt)
    gm = ge[1:R + 3]
    y0 = jnp.where((gm >= 0) & (gm < ng), y0, jnp.zeros((), cdt))

    acc1 = (jnp.dot(xc[1:R + 1], w1b_ref[0:CG], preferred_element_type=f32)
            + jnp.dot(xc[2:R + 2], w1b_ref[CG:2 * CG],
                      preferred_element_type=f32)
            + jnp.dot(xc[3:R + 3], w1b_ref[2 * CG:3 * CG],
                      preferred_element_type=f32)
            + jnp.dot(y0[0:R], w1b_ref[3 * CG:3 * CG + KG],
                      preferred_element_type=f32)
            + jnp.dot(y0[1:R + 1], w1b_ref[3 * CG + KG:3 * CG + 2 * KG],
                      preferred_element_type=f32)
            + jnp.dot(y0[2:R + 2], w1b_ref[3 * CG + 2 * KG:],
                      preferred_element_type=f32))
    y1 = jnp.maximum(acc1, 0.0).astype(cdt)

    z = (jnp.dot(xc[2:2 + R], whb_ref[0:CG], preferred_element_type=f32)
         + jnp.dot(y1, whb_ref[CG:], preferred_element_type=f32))
    out_ref[...] = jax.nn.softplus(z)


def kernel(inputs, W0, b0, W1, b1, Wh, bh):
    x = inputs[0]
    n, f = x.shape
    nb = n // T
    ng = n // G
    cdt = jnp.bfloat16

    W0t = W0.at[:, F:2 * F].add(-W0[:, :F])
    W1t = W1.at[:, F:2 * F].add(-W1[:, :F])
    Wht = Wh.at[F:2 * F].add(-Wh[:F])

    xg = jnp.concatenate(
        [x.astype(cdt),
         jnp.ones((n, 1), cdt),
         jnp.zeros((n, FP - f - 1), cdt)], axis=1).reshape(ng, G * FP)

    pad_p = lambda w: jnp.pad(w, ((0, 0), (0, FP - F), (0, 0)))
    w0b = _band_x(pad_p(W0t[:, :F]), pad_p(W0t[:, F:2 * F]),
                  3, 1, b0).astype(cdt)
    w1b = jnp.concatenate(
        [_band_x(pad_p(W1t[:, :F]), pad_p(W1t[:, F:2 * F]), 5, 2, b1),
         _band_y(W1t[:, 2 * F:], 5, 2)], axis=0).astype(cdt)
    pad_h = lambda w: jnp.pad(w, ((0, FP - F), (0, 0)))
    whb = _head_w(pad_h(Wht[:F]), pad_h(Wht[F:2 * F]), Wht[2 * F:],
                  bh).astype(cdt)

    full = lambda shape: pl.BlockSpec(shape, lambda i: (0,) * len(shape))
    out = pl.pallas_call(
        _fused_kernel,
        grid=(nb,),
        in_specs=[
            pl.BlockSpec((R, G * FP),
                         lambda i: (jnp.maximum(i - 1, 0), 0)),
            pl.BlockSpec((R, G * FP), lambda i: (i, 0)),
            pl.BlockSpec((R, G * FP),
                         lambda i: (jnp.minimum(i + 1, nb - 1), 0)),
            full((3 * CG, KG)), full((3 * CG + 3 * KG, KG)),
            full((CG + KG, G * BINS)),
        ],
        out_specs=pl.BlockSpec((R, G * BINS), lambda i: (i, 0)),
        out_shape=jax.ShapeDtypeStruct((ng, G * BINS), jnp.float32),
        compiler_params=pltpu.CompilerParams(
            dimension_semantics=("parallel",)),
    )(xg, xg, xg, w0b, w1b, whb)
    return out.reshape(n, BINS)[None]


# speedup vs baseline: 1.0063x; 1.0063x over previous
import jax
import jax.numpy as jnp
from jax.experimental import pallas as pl
from jax.experimental.pallas import tpu as pltpu

F = 51
FP = 64
K = 128
BINS = 2
G = 2
T = 8192
R = T // G
CG = 2 * G * FP
KG = G * K


def _band_x(Wp0, Wp1, ksize, ctr, bias):
    zero = jnp.zeros((FP, K), Wp0.dtype)
    oblocks = []
    for o in (-1, 0, 1):
        cols = []
        for s in range(G):
            rows = []
            for part, Wp in ((0, Wp0), (1, Wp1)):
                for r in range(G):
                    t = G * o + r - s + ctr
                    if 0 <= t < ksize:
                        piece = Wp[t]
                        if part == 0 and o == 0 and r == s:
                            piece = piece.at[F].set(bias)
                    else:
                        piece = zero
                    rows.append(piece)
            cols.append(jnp.concatenate(rows, axis=0))
        oblocks.append(jnp.concatenate(cols, axis=1))
    return jnp.concatenate(oblocks, axis=0)


def _band_y(Wy, ksize, ctr):
    zero = jnp.zeros((K, K), Wy.dtype)
    oblocks = []
    for o in (-1, 0, 1):
        cols = []
        for s in range(G):
            rows = []
            for r in range(G):
                t = G * o + r - s + ctr
                rows.append(Wy[t] if 0 <= t < ksize else zero)
            cols.append(jnp.concatenate(rows, axis=0))
        oblocks.append(jnp.concatenate(cols, axis=1))
    return jnp.concatenate(oblocks, axis=0)


def _head_w(Whp0, Whp1, Why, bias):
    zf = jnp.zeros((FP, BINS), Whp0.dtype)
    zy = jnp.zeros((K, BINS), Why.dtype)
    cols = []
    for s in range(G):
        rows = []
        for part, Wp in ((0, Whp0), (1, Whp1)):
            for r in range(G):
                if r == s:
                    piece = Wp.at[F].set(bias) if part == 0 else Wp
                else:
                    piece = zf
                rows.append(piece)
        for r in range(G):
            rows.append(Why if r == s else zy)
        cols.append(jnp.concatenate(rows, axis=0))
    return jnp.concatenate(cols, axis=1)


def _fused_kernel(prev_ref, cur_ref, next_ref,
                  w0b_ref, w1b_ref, whb_ref,
                  out_ref):
    i = pl.program_id(0)
    nb = pl.num_programs(0)
    f32 = jnp.float32
    cdt = cur_ref.dtype
    xe = jnp.concatenate(
        [prev_ref[R - 2:, :], cur_ref[...], next_ref[:2, :]], axis=0)
    ng = nb * R
    ge = i * R - 2 + jax.lax.broadcasted_iota(jnp.int32, (R + 4, 1), 0)
    xe = jnp.where((ge >= 0) & (ge < ng), xe, 0.0)
    xc = jnp.concatenate([xe, jnp.maximum(xe, 0.0)], axis=1)

    acc0 = (jnp.dot(xc[0:R + 2], w0b_ref[0:CG], preferred_element_type=f32)
            + jnp.dot(xc[1:R + 3], w0b_ref[CG:2 * CG],
                      preferred_element_type=f32)
            + jnp.dot(xc[2:R + 4], w0b_ref[2 * CG:3 * CG],
                      preferred_element_type=f32))
    y0 = jnp.maximum(acc0, 0.0).astype(cdt)
    gm = ge[1:R + 3]
    y0 = jnp.where((gm >= 0) & (gm < ng), y0, jnp.zeros((), cdt))

    acc1 = (jnp.dot(xc[1:R + 1], w1b_ref[0:CG], preferred_element_type=f32)
            + jnp.dot(xc[2:R + 2], w1b_ref[CG:2 * CG],
                      preferred_element_type=f32)
            + jnp.dot(xc[3:R + 3], w1b_ref[2 * CG:3 * CG],
                      preferred_element_type=f32)
            + jnp.dot(y0[0:R], w1b_ref[3 * CG:3 * CG + KG],
                      preferred_element_type=f32)
            + jnp.dot(y0[1:R + 1], w1b_ref[3 * CG + KG:3 * CG + 2 * KG],
                      preferred_element_type=f32)
            + jnp.dot(y0[2:R + 2], w1b_ref[3 * CG + 2 * KG:],
                      preferred_element_type=f32))
    y1 = jnp.maximum(acc1, 0.0).astype(cdt)

    z = (jnp.dot(xc[2:2 + R], whb_ref[0:CG], preferred_element_type=f32)
         + jnp.dot(y1, whb_ref[CG:], preferred_element_type=f32))
    out_ref[...] = jax.nn.softplus(z)


def kernel(inputs, W0, b0, W1, b1, Wh, bh):
    x = inputs[0]
    n, f = x.shape
    nb = n // T
    ng = n // G
    cdt = jnp.bfloat16

    W0t = W0.at[:, F:2 * F].add(-W0[:, :F])
    W1t = W1.at[:, F:2 * F].add(-W1[:, :F])
    Wht = Wh.at[F:2 * F].add(-Wh[:F])

    xg = jnp.concatenate(
        [x.astype(cdt),
         jnp.ones((n, 1), cdt),
         jnp.zeros((n, FP - f - 1), cdt)], axis=1).reshape(ng, G * FP)

    pad_p = lambda w: jnp.pad(w, ((0, 0), (0, FP - F), (0, 0)))
    w0b = _band_x(pad_p(W0t[:, :F]), pad_p(W0t[:, F:2 * F]),
                  3, 1, b0).astype(cdt)
    w1b = jnp.concatenate(
        [_band_x(pad_p(W1t[:, :F]), pad_p(W1t[:, F:2 * F]), 5, 2, b1),
         _band_y(W1t[:, 2 * F:], 5, 2)], axis=0).astype(cdt)
    pad_h = lambda w: jnp.pad(w, ((0, FP - F), (0, 0)))
    whb = _head_w(pad_h(Wht[:F]), pad_h(Wht[F:2 * F]), Wht[2 * F:],
                  bh).astype(cdt)

    full = lambda shape: pl.BlockSpec(shape, lambda i: (0,) * len(shape))
    out = pl.pallas_call(
        _fused_kernel,
        grid=(nb,),
        in_specs=[
            pl.BlockSpec((R, G * FP),
                         lambda i: (jnp.maximum(i - 1, 0), 0)),
            pl.BlockSpec((R, G * FP), lambda i: (i, 0)),
            pl.BlockSpec((R, G * FP),
                         lambda i: (jnp.minimum(i + 1, nb - 1), 0)),
            full((3 * CG, KG)), full((3 * CG + 3 * KG, KG)),
            full((CG + KG, G * BINS)),
        ],
        out_specs=pl.BlockSpec((R, G * BINS), lambda i: (i, 0)),
        out_shape=jax.ShapeDtypeStruct((ng, G * BINS), jnp.float32),
        compiler_params=pltpu.CompilerParams(
            dimension_semantics=("parallel",)),
    )(xg, xg, xg, w0b, w1b, whb)
    return out.reshape(n, BINS)[None]
